# baseline (speedup 1.0000x reference)
import functools

import jax
import jax.numpy as jnp
from jax import lax
from jax.experimental import pallas as pl
from jax.experimental.pallas import tpu as pltpu
from jax.experimental.pallas import tpu_sc as plsc

N_ATOMS = 10000
N_EDGES = 320000
D = 128
N_RBF = 50

NC = 2
NS = 16
NW = NC * NS
E_PER_W = N_EDGES // 2 // NW
CHUNK = 96
N_FULL = E_PER_W // CHUNK
TAIL = E_PER_W - N_FULL * CHUNK
ACC_ROWS = 10112
ROWS_PER_TILE = ACC_ROWS // NS
ZROWS = CHUNK

_LOG2 = 0.6931471805599453


_LOG2E = 1.4426950408889634


def _ssp(v):
    t = jnp.minimum(v * _LOG2E, 127.0)
    return (jnp.log2(jnp.exp2(t) + 1.0) - 1.0) * _LOG2


def _h_body(x_ref, w_ref, o_ref):
    o_ref[...] = jnp.dot(x_ref[...], w_ref[...],
                         preferred_element_type=jnp.float32)


def _compute_h(x, W_in2f):
    blk = 2000
    grid = N_ATOMS // blk
    return pl.pallas_call(
        _h_body,
        grid=(grid,),
        in_specs=[
            pl.BlockSpec((blk, D), lambda i: (i, 0)),
            pl.BlockSpec((D, D), lambda i: (0, 0)),
        ],
        out_specs=pl.BlockSpec((blk, D), lambda i: (i, 0)),
        out_shape=jax.ShapeDtypeStruct((N_ATOMS, D), jnp.float32),
    )(x, W_in2f)


def _wij_body(ft_ref, rc_ref, w1_ref, b1_ref, w2_ref, b2_ref, o_ref):
    t = lax.dot_general(ft_ref[...], w1_ref[...],
                        (((0,), (0,)), ((), ())),
                        preferred_element_type=jnp.float32)
    t = _ssp(t + b1_ref[...])
    w = jnp.dot(t, w2_ref[...], preferred_element_type=jnp.float32) + b2_ref[...]
    o_ref[...] = w * rc_ref[0, 0, :][:, None]


E_HALF = N_EDGES // 2


def _compute_wij(f_t, rcut2, W_f1, b_f1, W_f2, b_f2, half):
    blk = 16000
    grid = E_HALF // blk
    off = half * grid
    return pl.pallas_call(
        _wij_body,
        grid=(grid,),
        in_specs=[
            pl.BlockSpec((N_RBF, blk), lambda i: (0, i + off)),
            pl.BlockSpec((1, 1, blk), lambda i: (i + off, 0, 0)),
            pl.BlockSpec((N_RBF, D), lambda i: (0, 0)),
            pl.BlockSpec((D,), lambda i: (0,)),
            pl.BlockSpec((D, D), lambda i: (0, 0)),
            pl.BlockSpec((D,), lambda i: (0,)),
        ],
        out_specs=pl.BlockSpec((blk, D), lambda i: (i, 0)),
        out_shape=jax.ShapeDtypeStruct((E_HALF, D), jnp.float32),
    )(f_t, rcut2, W_f1, b_f1, W_f2, b_f2)


def _sc_body(h_hbm, wij_hbm, idxi_hbm, idxj_hbm, out_hbm,
             acc, xj0, xj1, w0, w1, idxj0, idxi0, idxj1, idxi1,
             idxj_t, idxi_t,
             semg0, semg1, semij0, semij1, semii0, semii1,
             semw0, semw1, semsc0, semsc1):
    c = lax.axis_index("c")
    s = lax.axis_index("s")
    wid = c * NS + s
    base = wid * E_PER_W
    xj = (xj0, xj1)
    w_v = (w0, w1)
    idxj = (idxj0, idxj1)
    idxi = (idxi0, idxi1)
    semg = (semg0, semg1)
    semij = (semij0, semij1)
    semii = (semii0, semii1)
    semw = (semw0, semw1)
    semsc = (semsc0, semsc1)
    last = N_FULL - 1

    def issue_idxj(k, p):
        off = base + k * CHUNK
        pltpu.async_copy(idxj_hbm.at[pl.ds(off, CHUNK)], idxj[p], semij[p])

    def wait_idxj(p):
        pltpu.make_async_copy(idxj_hbm.at[pl.ds(base, CHUNK)], idxj[p], semij[p]).wait()

    def issue_idxi(k, p):
        off = base + k * CHUNK
        pltpu.async_copy(idxi_hbm.at[pl.ds(off, CHUNK)], idxi[p], semii[p])

    def wait_idxi(p):
        pltpu.make_async_copy(idxi_hbm.at[pl.ds(base, CHUNK)], idxi[p], semii[p]).wait()

    def issue_wij(k, p):
        off = base + k * CHUNK
        pltpu.async_copy(wij_hbm.at[pl.ds(off, CHUNK)], w_v[p], semw[p])

    def wait_wij(p):
        pltpu.make_async_copy(wij_hbm.at[pl.ds(base, CHUNK)], w_v[p], semw[p]).wait()

    def issue_gather(p):
        pltpu.async_copy(h_hbm.at[idxj[p]], xj[p], semg[p])

    def wait_gather(p):
        pltpu.make_async_copy(h_hbm.at[idxj[p]], xj[p], semg[p]).wait()

    def issue_scatter(p):
        pltpu.async_copy(xj[p], acc.at[idxi[p]], semsc[p], add=True)

    def wait_scatter(p):
        pltpu.make_async_copy(xj[p], acc.at[idxi[p]], semsc[p]).wait()

    def mul(buf, wbuf, n):
        @plsc.parallel_loop(0, n, 1, unroll=4)
        def body(e):
            for l in range(D // 16):
                sl = pl.ds(l * 16, 16)
                buf[e, sl] = buf[e, sl] * wbuf[e, sl]

    issue_idxj(0, 0)
    issue_idxj(1, 1)
    issue_idxi(0, 0)
    issue_wij(0, 0)
    issue_wij(1, 1)

    def _zfill(e, _):
        for l in range(D // 16):
            xj0[e, pl.ds(l * 16, 16)] = jnp.zeros((16,), jnp.float32)
        return 0
    row0 = s * ROWS_PER_TILE
    nfull_z = ROWS_PER_TILE // ZROWS
    zrem = ROWS_PER_TILE - nfull_z * ZROWS
    lax.fori_loop(0, ZROWS, _zfill, 0)
    for kk in range(nfull_z):
        pltpu.async_copy(xj0, acc.at[pl.ds(row0 + kk * ZROWS, ZROWS)], semsc0)
    if zrem:
        pltpu.async_copy(xj0.at[pl.ds(0, zrem)],
                         acc.at[pl.ds(row0 + nfull_z * ZROWS, zrem)], semsc0)
    for kk in range(nfull_z):
        pltpu.make_async_copy(xj0, acc.at[pl.ds(row0 + kk * ZROWS, ZROWS)],
                              semsc0).wait()
    if zrem:
        pltpu.make_async_copy(xj0.at[pl.ds(0, zrem)],
                              acc.at[pl.ds(row0 + nfull_z * ZROWS, zrem)],
                              semsc0).wait()
    plsc.subcore_barrier()

    wait_idxj(0)
    issue_gather(0)

    def half(i, k, p, guarded):
        q = 1 - p
        wait_gather(p)
        issue_idxj(jnp.minimum(k + 2, last), p)

        def _free_q():
            wait_scatter(q)
        if guarded:
            pl.when(i > 0)(_free_q)
        else:
            _free_q()
        issue_idxi(jnp.minimum(k + 1, last), q)
        wait_idxj(q)
        issue_gather(q)
        wait_wij(p)
        mul(xj[p], w_v[p], CHUNK)
        issue_wij(jnp.minimum(k + 2, last), p)
        wait_idxi(p)
        issue_scatter(p)

    def body(i, _):
        half(i, 2 * i, 0, True)
        half(i, 2 * i + 1, 1, False)
        return 0
    lax.fori_loop(0, N_FULL // 2, body, 0)

    wait_gather(0)
    wait_idxj(1)
    wait_idxi(0)
    wait_wij(0)
    wait_wij(1)
    wait_scatter(1)

    off = base + N_FULL * CHUNK
    pltpu.sync_copy(idxj_hbm.at[pl.ds(off, TAIL)], idxj_t)
    pltpu.sync_copy(idxi_hbm.at[pl.ds(off, TAIL)], idxi_t)
    pltpu.async_copy(h_hbm.at[idxj_t], xj0.at[pl.ds(0, TAIL)], semg0).wait()
    pltpu.sync_copy(wij_hbm.at[pl.ds(off, TAIL)], w0.at[pl.ds(0, TAIL)])
    mul(xj0, w0, TAIL)
    pltpu.sync_copy(xj0.at[pl.ds(0, TAIL)], acc.at[idxi_t], add=True)

    plsc.subcore_barrier()
    pltpu.sync_copy(acc.at[pl.ds(row0, ROWS_PER_TILE)],
                    out_hbm.at[c, pl.ds(row0, ROWS_PER_TILE)])


def _sc_aggregate(h, wij, idx_i, idx_j):
    mesh = plsc.VectorSubcoreMesh(core_axis_name="c", subcore_axis_name="s")
    f = pl.kernel(
        _sc_body,
        out_type=jax.ShapeDtypeStruct((NC, ACC_ROWS, D), jnp.float32),
        mesh=mesh,
        scratch_types=[
            pltpu.VMEM_SHARED((ACC_ROWS, D), jnp.float32),
            pltpu.VMEM((CHUNK, D), jnp.float32),
            pltpu.VMEM((CHUNK, D), jnp.float32),
            pltpu.VMEM((CHUNK, D), jnp.float32),
            pltpu.VMEM((CHUNK, D), jnp.float32),
            pltpu.VMEM((CHUNK,), jnp.int32),
            pltpu.VMEM((CHUNK,), jnp.int32),
            pltpu.VMEM((CHUNK,), jnp.int32),
            pltpu.VMEM((CHUNK,), jnp.int32),
            pltpu.VMEM((TAIL,), jnp.int32),
            pltpu.VMEM((TAIL,), jnp.int32),
            pltpu.SemaphoreType.DMA,
            pltpu.SemaphoreType.DMA,
            pltpu.SemaphoreType.DMA,
            pltpu.SemaphoreType.DMA,
            pltpu.SemaphoreType.DMA,
            pltpu.SemaphoreType.DMA,
            pltpu.SemaphoreType.DMA,
            pltpu.SemaphoreType.DMA,
            pltpu.SemaphoreType.DMA,
            pltpu.SemaphoreType.DMA,
        ],
    )
    return f(h, wij, idx_i, idx_j)


def _out_body(p0_ref, p1_ref, p2_ref, p3_ref, w1_ref, b1_ref, w2_ref, b2_ref, o_ref):
    agg = (p0_ref[0] + p1_ref[0]) + (p2_ref[0] + p3_ref[0])
    t = _ssp(jnp.dot(agg, w1_ref[...],
                     preferred_element_type=jnp.float32) + b1_ref[...])
    o_ref[...] = jnp.dot(t, w2_ref[...],
                         preferred_element_type=jnp.float32) + b2_ref[...]


def _compute_out(pA, pB, W_o1, b_o1, W_o2, b_o2):
    blk = 2000
    grid = N_ATOMS // blk
    return pl.pallas_call(
        _out_body,
        grid=(grid,),
        in_specs=[
            pl.BlockSpec((1, blk, D), lambda i: (0, i, 0)),
            pl.BlockSpec((1, blk, D), lambda i: (1, i, 0)),
            pl.BlockSpec((1, blk, D), lambda i: (0, i, 0)),
            pl.BlockSpec((1, blk, D), lambda i: (1, i, 0)),
            pl.BlockSpec((D, D), lambda i: (0, 0)),
            pl.BlockSpec((D,), lambda i: (0,)),
            pl.BlockSpec((D, D), lambda i: (0, 0)),
            pl.BlockSpec((D,), lambda i: (0,)),
        ],
        out_specs=pl.BlockSpec((blk, D), lambda i: (i, 0)),
        out_shape=jax.ShapeDtypeStruct((N_ATOMS, D), jnp.float32),
    )(pA, pA, pB, pB, W_o1, b_o1, W_o2, b_o2)


def kernel(x, f_ij, idx_i, idx_j, rcut_ij,
           W_in2f, W_f1, b_f1, W_f2, b_f2, W_o1, b_o1, W_o2, b_o2):
    idx_i = idx_i.astype(jnp.int32)
    idx_j = idx_j.astype(jnp.int32)
    f_t = f_ij.T
    rcut2 = rcut_ij.reshape(N_EDGES // 16000, 1, 16000)
    h = _compute_h(x, W_in2f)
    wij_a = _compute_wij(f_t, rcut2, W_f1, b_f1, W_f2, b_f2, 0)
    pA = _sc_aggregate(h, wij_a, idx_i[:E_HALF], idx_j[:E_HALF])
    wij_b = _compute_wij(f_t, rcut2, W_f1, b_f1, W_f2, b_f2, 1)
    pB = _sc_aggregate(h, wij_b, idx_i[E_HALF:], idx_j[E_HALF:])
    return _compute_out(pA, pB, W_o1, b_o1, W_o2, b_o2)


# speedup vs baseline: 1.0120x; 1.0120x over previous
import functools

import jax
import jax.numpy as jnp
from jax import lax
from jax.experimental import pallas as pl
from jax.experimental.pallas import tpu as pltpu
from jax.experimental.pallas import tpu_sc as plsc

N_ATOMS = 10000
N_EDGES = 320000
D = 128
N_RBF = 50

NC = 2
NS = 16
NW = NC * NS
E_PER_W = N_EDGES // 2 // NW
CHUNK = 96
N_FULL = E_PER_W // CHUNK
TAIL = E_PER_W - N_FULL * CHUNK
ACC_ROWS = 10112
ROWS_PER_TILE = ACC_ROWS // NS
ZROWS = CHUNK

_LOG2 = 0.6931471805599453


_LOG2E = 1.4426950408889634


def _ssp(v):
    t = jnp.minimum(v * _LOG2E, 127.0)
    return (jnp.log2(jnp.exp2(t) + 1.0) - 1.0) * _LOG2


def _h_body(x_ref, w_ref, o_ref):
    o_ref[...] = jnp.dot(x_ref[...], w_ref[...],
                         preferred_element_type=jnp.float32)


def _compute_h(x, W_in2f):
    blk = 2000
    grid = N_ATOMS // blk
    return pl.pallas_call(
        _h_body,
        grid=(grid,),
        in_specs=[
            pl.BlockSpec((blk, D), lambda i: (i, 0)),
            pl.BlockSpec((D, D), lambda i: (0, 0)),
        ],
        out_specs=pl.BlockSpec((blk, D), lambda i: (i, 0)),
        out_shape=jax.ShapeDtypeStruct((N_ATOMS, D), jnp.float32),
    )(x, W_in2f)


def _wij_body(ft_ref, rc_ref, w1_ref, b1_ref, w2_ref, b2_ref, o_ref):
    t = lax.dot_general(ft_ref[...], w1_ref[...],
                        (((0,), (0,)), ((), ())),
                        preferred_element_type=jnp.float32)
    t = _ssp(t + b1_ref[...])
    w = jnp.dot(t, w2_ref[...], preferred_element_type=jnp.float32) + b2_ref[...]
    o_ref[...] = w * rc_ref[0, 0, :][:, None]


E_HALF = N_EDGES // 2


def _compute_wij(f_t, rcut2, W_f1, b_f1, W_f2, b_f2, half):
    blk = 16000
    grid = E_HALF // blk
    off = half * grid
    return pl.pallas_call(
        _wij_body,
        grid=(grid,),
        in_specs=[
            pl.BlockSpec((N_RBF, blk), lambda i: (0, i + off)),
            pl.BlockSpec((1, 1, blk), lambda i: (i + off, 0, 0)),
            pl.BlockSpec((N_RBF, D), lambda i: (0, 0)),
            pl.BlockSpec((D,), lambda i: (0,)),
            pl.BlockSpec((D, D), lambda i: (0, 0)),
            pl.BlockSpec((D,), lambda i: (0,)),
        ],
        out_specs=pl.BlockSpec((blk, D), lambda i: (i, 0)),
        out_shape=jax.ShapeDtypeStruct((E_HALF, D), jnp.float32),
    )(f_t, rcut2, W_f1, b_f1, W_f2, b_f2)


def _sc_body(h_hbm, wij_hbm, idxi_hbm, idxj_hbm, out_hbm,
             acc, xj0, xj1, w0, w1, idxj0, idxi0, idxj1, idxi1,
             idxj_t, idxi_t,
             semg0, semg1, semij0, semij1, semii0, semii1,
             semw0, semw1, semsc0, semsc1):
    c = lax.axis_index("c")
    s = lax.axis_index("s")
    wid = c * NS + s
    base = wid * E_PER_W
    xj = (xj0, xj1)
    w_v = (w0, w1)
    idxj = (idxj0, idxj1)
    idxi = (idxi0, idxi1)
    semg = (semg0, semg1)
    semij = (semij0, semij1)
    semii = (semii0, semii1)
    semw = (semw0, semw1)
    semsc = (semsc0, semsc1)
    last = N_FULL - 1

    def issue_idxj(k, p):
        off = base + k * CHUNK
        pltpu.async_copy(idxj_hbm.at[pl.ds(off, CHUNK)], idxj[p], semij[p])

    def wait_idxj(p):
        pltpu.make_async_copy(idxj_hbm.at[pl.ds(base, CHUNK)], idxj[p], semij[p]).wait()

    def issue_idxi(k, p):
        off = base + k * CHUNK
        pltpu.async_copy(idxi_hbm.at[pl.ds(off, CHUNK)], idxi[p], semii[p])

    def wait_idxi(p):
        pltpu.make_async_copy(idxi_hbm.at[pl.ds(base, CHUNK)], idxi[p], semii[p]).wait()

    def issue_wij(k, p):
        off = base + k * CHUNK
        pltpu.async_copy(wij_hbm.at[pl.ds(off, CHUNK)], w_v[p], semw[p])

    def wait_wij(p):
        pltpu.make_async_copy(wij_hbm.at[pl.ds(base, CHUNK)], w_v[p], semw[p]).wait()

    def issue_gather(p):
        pltpu.async_copy(h_hbm.at[idxj[p]], xj[p], semg[p])

    def wait_gather(p):
        pltpu.make_async_copy(h_hbm.at[idxj[p]], xj[p], semg[p]).wait()

    def issue_scatter(p):
        pltpu.async_copy(xj[p], acc.at[idxi[p]], semsc[p], add=True)

    def wait_scatter(p):
        pltpu.make_async_copy(xj[p], acc.at[idxi[p]], semsc[p]).wait()

    def mul(buf, wbuf, n):
        def body(e2, _):
            for j in range(2):
                e = e2 * 2 + j
                for l in range(D // 16):
                    sl = pl.ds(l * 16, 16)
                    buf[e, sl] = buf[e, sl] * wbuf[e, sl]
            return 0
        lax.fori_loop(0, n // 2, body, 0)

    issue_idxj(0, 0)
    issue_idxj(1, 1)
    issue_idxi(0, 0)
    issue_wij(0, 0)
    issue_wij(1, 1)

    def _zfill(e, _):
        for l in range(D // 16):
            xj0[e, pl.ds(l * 16, 16)] = jnp.zeros((16,), jnp.float32)
        return 0
    row0 = s * ROWS_PER_TILE
    nfull_z = ROWS_PER_TILE // ZROWS
    zrem = ROWS_PER_TILE - nfull_z * ZROWS
    lax.fori_loop(0, ZROWS, _zfill, 0)
    for kk in range(nfull_z):
        pltpu.async_copy(xj0, acc.at[pl.ds(row0 + kk * ZROWS, ZROWS)], semsc0)
    if zrem:
        pltpu.async_copy(xj0.at[pl.ds(0, zrem)],
                         acc.at[pl.ds(row0 + nfull_z * ZROWS, zrem)], semsc0)
    for kk in range(nfull_z):
        pltpu.make_async_copy(xj0, acc.at[pl.ds(row0 + kk * ZROWS, ZROWS)],
                              semsc0).wait()
    if zrem:
        pltpu.make_async_copy(xj0.at[pl.ds(0, zrem)],
                              acc.at[pl.ds(row0 + nfull_z * ZROWS, zrem)],
                              semsc0).wait()
    plsc.subcore_barrier()

    wait_idxj(0)
    issue_gather(0)

    def half(i, k, p, guarded):
        q = 1 - p
        wait_gather(p)
        issue_idxj(jnp.minimum(k + 2, last), p)

        def _free_q():
            wait_scatter(q)
        if guarded:
            pl.when(i > 0)(_free_q)
        else:
            _free_q()
        issue_idxi(jnp.minimum(k + 1, last), q)
        wait_idxj(q)
        issue_gather(q)
        wait_wij(p)
        mul(xj[p], w_v[p], CHUNK)
        issue_wij(jnp.minimum(k + 2, last), p)
        wait_idxi(p)
        issue_scatter(p)

    def body(i, _):
        half(i, 2 * i, 0, True)
        half(i, 2 * i + 1, 1, False)
        return 0
    lax.fori_loop(0, N_FULL // 2, body, 0)

    wait_gather(0)
    wait_idxj(1)
    wait_idxi(0)
    wait_wij(0)
    wait_wij(1)
    wait_scatter(1)

    off = base + N_FULL * CHUNK
    pltpu.sync_copy(idxj_hbm.at[pl.ds(off, TAIL)], idxj_t)
    pltpu.sync_copy(idxi_hbm.at[pl.ds(off, TAIL)], idxi_t)
    pltpu.async_copy(h_hbm.at[idxj_t], xj0.at[pl.ds(0, TAIL)], semg0).wait()
    pltpu.sync_copy(wij_hbm.at[pl.ds(off, TAIL)], w0.at[pl.ds(0, TAIL)])
    mul(xj0, w0, TAIL)
    pltpu.sync_copy(xj0.at[pl.ds(0, TAIL)], acc.at[idxi_t], add=True)

    plsc.subcore_barrier()
    pltpu.sync_copy(acc.at[pl.ds(row0, ROWS_PER_TILE)],
                    out_hbm.at[c, pl.ds(row0, ROWS_PER_TILE)])


def _sc_aggregate(h, wij, idx_i, idx_j):
    mesh = plsc.VectorSubcoreMesh(core_axis_name="c", subcore_axis_name="s")
    f = pl.kernel(
        _sc_body,
        out_type=jax.ShapeDtypeStruct((NC, ACC_ROWS, D), jnp.float32),
        mesh=mesh,
        scratch_types=[
            pltpu.VMEM_SHARED((ACC_ROWS, D), jnp.float32),
            pltpu.VMEM((CHUNK, D), jnp.float32),
            pltpu.VMEM((CHUNK, D), jnp.float32),
            pltpu.VMEM((CHUNK, D), jnp.float32),
            pltpu.VMEM((CHUNK, D), jnp.float32),
            pltpu.VMEM((CHUNK,), jnp.int32),
            pltpu.VMEM((CHUNK,), jnp.int32),
            pltpu.VMEM((CHUNK,), jnp.int32),
            pltpu.VMEM((CHUNK,), jnp.int32),
            pltpu.VMEM((TAIL,), jnp.int32),
            pltpu.VMEM((TAIL,), jnp.int32),
            pltpu.SemaphoreType.DMA,
            pltpu.SemaphoreType.DMA,
            pltpu.SemaphoreType.DMA,
            pltpu.SemaphoreType.DMA,
            pltpu.SemaphoreType.DMA,
            pltpu.SemaphoreType.DMA,
            pltpu.SemaphoreType.DMA,
            pltpu.SemaphoreType.DMA,
            pltpu.SemaphoreType.DMA,
            pltpu.SemaphoreType.DMA,
        ],
    )
    return f(h, wij, idx_i, idx_j)


def _out_body(p0_ref, p1_ref, p2_ref, p3_ref, w1_ref, b1_ref, w2_ref, b2_ref, o_ref):
    agg = (p0_ref[0] + p1_ref[0]) + (p2_ref[0] + p3_ref[0])
    t = _ssp(jnp.dot(agg, w1_ref[...],
                     preferred_element_type=jnp.float32) + b1_ref[...])
    o_ref[...] = jnp.dot(t, w2_ref[...],
                         preferred_element_type=jnp.float32) + b2_ref[...]


def _compute_out(pA, pB, W_o1, b_o1, W_o2, b_o2):
    blk = 2000
    grid = N_ATOMS // blk
    return pl.pallas_call(
        _out_body,
        grid=(grid,),
        in_specs=[
            pl.BlockSpec((1, blk, D), lambda i: (0, i, 0)),
            pl.BlockSpec((1, blk, D), lambda i: (1, i, 0)),
            pl.BlockSpec((1, blk, D), lambda i: (0, i, 0)),
            pl.BlockSpec((1, blk, D), lambda i: (1, i, 0)),
            pl.BlockSpec((D, D), lambda i: (0, 0)),
            pl.BlockSpec((D,), lambda i: (0,)),
            pl.BlockSpec((D, D), lambda i: (0, 0)),
            pl.BlockSpec((D,), lambda i: (0,)),
        ],
        out_specs=pl.BlockSpec((blk, D), lambda i: (i, 0)),
        out_shape=jax.ShapeDtypeStruct((N_ATOMS, D), jnp.float32),
    )(pA, pA, pB, pB, W_o1, b_o1, W_o2, b_o2)


def kernel(x, f_ij, idx_i, idx_j, rcut_ij,
           W_in2f, W_f1, b_f1, W_f2, b_f2, W_o1, b_o1, W_o2, b_o2):
    idx_i = idx_i.astype(jnp.int32)
    idx_j = idx_j.astype(jnp.int32)
    f_t = f_ij.T
    rcut2 = rcut_ij.reshape(N_EDGES // 16000, 1, 16000)
    h = _compute_h(x, W_in2f)
    wij_a = _compute_wij(f_t, rcut2, W_f1, b_f1, W_f2, b_f2, 0)
    pA = _sc_aggregate(h, wij_a, idx_i[:E_HALF], idx_j[:E_HALF])
    wij_b = _compute_wij(f_t, rcut2, W_f1, b_f1, W_f2, b_f2, 1)
    pB = _sc_aggregate(h, wij_b, idx_i[E_HALF:], idx_j[E_HALF:])
    return _compute_out(pA, pB, W_o1, b_o1, W_o2, b_o2)
